# baseline (speedup 1.0000x reference)
import functools

import jax
import jax.numpy as jnp
from jax import lax
from jax.experimental import pallas as pl
from jax.experimental.pallas import tpu as pltpu
from jax.experimental.pallas import tpu_sc as plsc

N_MOL = 4096

_B = 12800
_NB = 25

_CORES = 2
_TILES = 16
_ROWS = 80
_CHUNK = 128
_NPAD = _CORES * _TILES * _ROWS * _CHUNK


_HALF = _B // 2


def _mv_body(w_ref, b_ref, xa_ref, xb_ref, o_ref):
    dn = (((1,), (1,)), ((), ()))
    ra = lax.dot_general(
        w_ref[...], xa_ref[0], dn, preferred_element_type=jnp.float32
    )
    rb = lax.dot_general(
        w_ref[...], xb_ref[0], dn, preferred_element_type=jnp.float32
    )
    b0 = b_ref[0, 0]
    o_ref[0, :, :_HALF] = ra + b0
    o_ref[0, :, _HALF:] = rb + b0


def _matvec(x, w0, b0):
    x3 = x.reshape(_NB, _B, 128)
    out = pl.pallas_call(
        _mv_body,
        grid=(_NB,),
        in_specs=[
            pl.BlockSpec((1, 128), lambda i: (0, 0)),
            pl.BlockSpec(memory_space=pltpu.SMEM),
            pl.BlockSpec((1, _HALF, 128), lambda i: (i, 0, 0)),
            pl.BlockSpec((1, _HALF, 128), lambda i: (i, 1, 0)),
        ],
        out_specs=pl.BlockSpec((1, 1, _B), lambda i: (i, 0, 0)),
        out_shape=jax.ShapeDtypeStruct((_NB, 1, _B), jnp.float32),
    )(w0, b0, x3, x3)
    return out.reshape(_NB * _B)


def _segsum(v_pad, idx_pad):
    mesh = plsc.VectorSubcoreMesh(
        core_axis_name="c", subcore_axis_name="s", num_cores=_CORES
    )

    @functools.partial(
        pl.kernel,
        out_type=jax.ShapeDtypeStruct((_CORES, N_MOL), jnp.float32),
        mesh=mesh,
        scratch_types=[
            pltpu.VMEM((_ROWS, _CHUNK), jnp.int32),
            pltpu.VMEM((_ROWS, _CHUNK), jnp.float32),
            pltpu.VMEM((N_MOL // _TILES,), jnp.float32),
            pltpu.VMEM_SHARED((N_MOL,), jnp.float32),
            pltpu.SemaphoreType.DMA,
        ],
    )
    def seg(v_hbm, idx_hbm, out_hbm, idx_v, val_v, z_v, acc_sh, sem):
        cid = lax.axis_index("c")
        sid = lax.axis_index("s")
        wid = cid * _TILES + sid
        piece = N_MOL // _TILES

        def zero_body(i, _):
            z_v[pl.ds(i * 16, 16)] = jnp.zeros((16,), jnp.float32)
            return 0

        lax.fori_loop(0, piece // 16, zero_body, 0)
        pltpu.sync_copy(z_v, acc_sh.at[pl.ds(sid * piece, piece)])
        plsc.subcore_barrier()

        pltpu.sync_copy(idx_hbm.at[wid], idx_v)
        pltpu.sync_copy(v_hbm.at[wid], val_v)

        def fire_body(j, _):
            pltpu.async_copy(val_v.at[j], acc_sh.at[idx_v.at[j]], sem, add=True)
            return 0

        lax.fori_loop(0, _ROWS, fire_body, 0)

        def drain_body(j, _):
            pltpu.make_async_copy(val_v.at[j], acc_sh.at[idx_v.at[j]], sem).wait()
            return 0

        lax.fori_loop(0, _ROWS, drain_body, 0)
        plsc.subcore_barrier()

        pltpu.sync_copy(
            acc_sh.at[pl.ds(sid * piece, piece)],
            out_hbm.at[cid].at[pl.ds(sid * piece, piece)],
        )

    return seg(v_pad, idx_pad)


def _combine_body(p_ref, o_ref):
    o_ref[...] = p_ref[0] + p_ref[1]


def _combine(partials):
    return pl.pallas_call(
        _combine_body,
        out_shape=jax.ShapeDtypeStruct((N_MOL,), jnp.float32),
    )(partials)


def kernel(scalar_representation, idx_m, W, b):
    x = scalar_representation
    n = x.shape[0]
    w0 = W[0:1, :]
    b0 = b[0].reshape(1, 1)

    v = _matvec(x, w0, b0)

    pad = _NPAD - n
    v_pad = jnp.pad(v, (0, pad)).reshape(_CORES * _TILES, _ROWS, _CHUNK)
    idx_pad = jnp.pad(idx_m.astype(jnp.int32), (0, pad)).reshape(
        _CORES * _TILES, _ROWS, _CHUNK
    )
    return _combine(_segsum(v_pad, idx_pad))


# speedup vs baseline: 1.0081x; 1.0081x over previous
import functools

import jax
import jax.numpy as jnp
from jax import lax
from jax.experimental import pallas as pl
from jax.experimental.pallas import tpu as pltpu
from jax.experimental.pallas import tpu_sc as plsc

N_MOL = 4096

_B = 20000
_NB = 16

_CORES = 2
_TILES = 16
_ROWS = 80
_CHUNK = 128
_NPAD = _CORES * _TILES * _ROWS * _CHUNK


_HALF = _B // 2


def _mv_body(w_ref, b_ref, xa_ref, xb_ref, o_ref):
    dn = (((1,), (1,)), ((), ()))
    ra = lax.dot_general(
        w_ref[...], xa_ref[0], dn, preferred_element_type=jnp.float32
    )
    rb = lax.dot_general(
        w_ref[...], xb_ref[0], dn, preferred_element_type=jnp.float32
    )
    b0 = b_ref[0, 0]
    o_ref[0, :, :_HALF] = ra + b0
    o_ref[0, :, _HALF:] = rb + b0


def _matvec(x, w0, b0):
    x3 = x.reshape(_NB, _B, 128)
    out = pl.pallas_call(
        _mv_body,
        grid=(_NB,),
        in_specs=[
            pl.BlockSpec((1, 128), lambda i: (0, 0)),
            pl.BlockSpec(memory_space=pltpu.SMEM),
            pl.BlockSpec((1, _HALF, 128), lambda i: (i, 0, 0)),
            pl.BlockSpec((1, _HALF, 128), lambda i: (i, 1, 0)),
        ],
        out_specs=pl.BlockSpec((1, 1, _B), lambda i: (i, 0, 0)),
        out_shape=jax.ShapeDtypeStruct((_NB, 1, _B), jnp.float32),
    )(w0, b0, x3, x3)
    return out.reshape(_NB * _B)


def _segsum(v_pad, idx_pad):
    mesh = plsc.VectorSubcoreMesh(
        core_axis_name="c", subcore_axis_name="s", num_cores=_CORES
    )

    @functools.partial(
        pl.kernel,
        out_type=jax.ShapeDtypeStruct((_CORES, N_MOL), jnp.float32),
        mesh=mesh,
        scratch_types=[
            pltpu.VMEM((_ROWS, _CHUNK), jnp.int32),
            pltpu.VMEM((_ROWS, _CHUNK), jnp.float32),
            pltpu.VMEM((N_MOL // _TILES,), jnp.float32),
            pltpu.VMEM_SHARED((N_MOL,), jnp.float32),
            pltpu.SemaphoreType.DMA,
        ],
    )
    def seg(v_hbm, idx_hbm, out_hbm, idx_v, val_v, z_v, acc_sh, sem):
        cid = lax.axis_index("c")
        sid = lax.axis_index("s")
        wid = cid * _TILES + sid
        piece = N_MOL // _TILES

        def zero_body(i, _):
            z_v[pl.ds(i * 16, 16)] = jnp.zeros((16,), jnp.float32)
            return 0

        lax.fori_loop(0, piece // 16, zero_body, 0)
        pltpu.sync_copy(z_v, acc_sh.at[pl.ds(sid * piece, piece)])
        plsc.subcore_barrier()

        pltpu.sync_copy(idx_hbm.at[wid], idx_v)
        pltpu.sync_copy(v_hbm.at[wid], val_v)

        def fire_body(j, _):
            pltpu.async_copy(val_v.at[j], acc_sh.at[idx_v.at[j]], sem, add=True)
            return 0

        lax.fori_loop(0, _ROWS, fire_body, 0)

        def drain_body(j, _):
            pltpu.make_async_copy(val_v.at[j], acc_sh.at[idx_v.at[j]], sem).wait()
            return 0

        lax.fori_loop(0, _ROWS, drain_body, 0)
        plsc.subcore_barrier()

        pltpu.sync_copy(
            acc_sh.at[pl.ds(sid * piece, piece)],
            out_hbm.at[cid].at[pl.ds(sid * piece, piece)],
        )

    return seg(v_pad, idx_pad)


def _combine_body(p_ref, o_ref):
    o_ref[...] = p_ref[0] + p_ref[1]


def _combine(partials):
    return pl.pallas_call(
        _combine_body,
        out_shape=jax.ShapeDtypeStruct((N_MOL,), jnp.float32),
    )(partials)


def kernel(scalar_representation, idx_m, W, b):
    x = scalar_representation
    n = x.shape[0]
    w0 = W[0:1, :]
    b0 = b[0].reshape(1, 1)

    v = _matvec(x, w0, b0)

    pad = _NPAD - n
    v_pad = jnp.pad(v, (0, pad)).reshape(_CORES * _TILES, _ROWS, _CHUNK)
    idx_pad = jnp.pad(idx_m.astype(jnp.int32), (0, pad)).reshape(
        _CORES * _TILES, _ROWS, _CHUNK
    )
    return _combine(_segsum(v_pad, idx_pad))
